# Initial kernel scaffold; baseline (speedup 1.0000x reference)
import jax
import jax.numpy as jnp
from jax.experimental import pallas as pl


def kernel(x, duration, max_len):
    raise NotImplementedError("write your pallas kernel here")


# speedup vs baseline: 26.8296x; 26.8296x over previous
import functools

import jax
import jax.numpy as jnp
from jax import lax
from jax.experimental import pallas as pl
from jax.experimental.pallas import tpu as pltpu
from jax.experimental.pallas import tpu_sc as plsc

_L = 16


@functools.partial(jax.jit, static_argnums=(2, 3, 4))
def _length_regulate(x2d, duration, B, T, D):
    ML = 2048
    info = plsc.get_sparse_core_info()
    NC, NS = info.num_cores, info.num_subcores
    NW = NC * NS
    SLOTS = NW // B
    F = ML // SLOTS
    CH = 64
    NCH = F // CH

    mesh = plsc.VectorSubcoreMesh(core_axis_name="c", subcore_axis_name="s")

    @functools.partial(
        pl.kernel,
        mesh=mesh,
        compiler_params=pltpu.CompilerParams(needs_layout_passes=False),
        out_type=[
            jax.ShapeDtypeStruct((B, ML, D), jnp.float32),
            jax.ShapeDtypeStruct((B, _L), jnp.int32),
        ],
        scratch_types=[
            pltpu.VMEM((T,), jnp.int32),
            pltpu.VMEM((ML,), jnp.int32),
            pltpu.VMEM((ML // CH, CH), jnp.int32),
            pltpu.VMEM((CH, D), jnp.float32),
            pltpu.VMEM((_L,), jnp.int32),
            pltpu.SemaphoreType.DMA,
        ],
    )
    def body(x_hbm, dur_hbm, out_hbm, mel_hbm, dur_v, cnt_v, idx_v, rows_v,
             mel_v, sem):
        wid = lax.axis_index("s") * NC + lax.axis_index("c")
        b = wid // SLOTS
        slot = wid % SLOTS

        pltpu.sync_copy(dur_hbm.at[b], dur_v)

        zeros_i = jnp.zeros((_L,), jnp.int32)
        for i in range(ML // _L):
            cnt_v[pl.ds(i * _L, _L)] = zeros_i

        ones_i = jnp.ones((_L,), jnp.int32)
        carry = jnp.int32(0)
        for i in range(T // _L):
            v = dur_v[pl.ds(i * _L, _L)]
            s = jnp.cumsum(v) + carry
            carry = carry + jnp.sum(v)
            in_range = s <= (ML - 1)
            ci = jnp.minimum(s, ML - 1)
            plsc.addupdate_scatter(cnt_v, [ci], ones_i, mask=in_range)
        total = carry

        base_row = b * T
        carry2 = jnp.int32(0)
        for i in range(ML // _L):
            v = cnt_v[pl.ds(i * _L, _L)]
            j = jnp.cumsum(v) + carry2
            carry2 = carry2 + jnp.sum(v)
            jc = jnp.minimum(j, T - 1)
            idx_v[i // (CH // _L), pl.ds((i % (CH // _L)) * _L, _L)] = (
                jc + base_row)

        @pl.when(slot == 0)
        def _():
            mel_v[...] = jnp.full((_L,), jnp.maximum(total, 1), jnp.int32)
            pltpu.sync_copy(mel_v, mel_hbm.at[b])

        zeros_f = jnp.zeros((_L,), jnp.float32)
        for c in range(NCH):
            t0 = slot * F + c * CH
            nv = jnp.clip(total - t0, 0, CH)

            @pl.when(nv > 0)
            def _():
                pltpu.async_copy(
                    x_hbm.at[idx_v.at[slot * NCH + c]], rows_v, sem).wait()

            def zero_row(r, _):
                for i in range(D // _L):
                    rows_v[r, pl.ds(i * _L, _L)] = zeros_f
                return _

            lax.fori_loop(nv, CH, zero_row, 0)
            pltpu.sync_copy(rows_v, out_hbm.at[b, pl.ds(t0, CH)])

    return body(x2d, duration)


def kernel(x, duration, max_len):
    B, T, D = x.shape
    out, mel2d = _length_regulate(
        x.reshape(B * T, D), duration.astype(jnp.int32), B, T, D)
    return out, mel2d[:, 0]


# speedup vs baseline: 28.2677x; 1.0536x over previous
import functools

import jax
import jax.numpy as jnp
from jax import lax
from jax.experimental import pallas as pl
from jax.experimental.pallas import tpu as pltpu
from jax.experimental.pallas import tpu_sc as plsc

_L = 16


@functools.partial(jax.jit, static_argnums=(2, 3, 4))
def _length_regulate(x2d, duration, B, T, D):
    ML = 2048
    info = plsc.get_sparse_core_info()
    NC, NS = info.num_cores, info.num_subcores
    NW = NC * NS
    SLOTS = NW // B
    F = ML // SLOTS
    CH = 64
    NCH = F // CH
    NBUF = 3

    mesh = plsc.VectorSubcoreMesh(core_axis_name="c", subcore_axis_name="s")

    @functools.partial(
        pl.kernel,
        mesh=mesh,
        compiler_params=pltpu.CompilerParams(needs_layout_passes=False),
        out_type=[
            jax.ShapeDtypeStruct((B, ML, D), jnp.float32),
            jax.ShapeDtypeStruct((B, _L), jnp.int32),
        ],
        scratch_types=[
            pltpu.VMEM((T,), jnp.int32),
            pltpu.VMEM((F,), jnp.int32),
            pltpu.VMEM((NCH, CH), jnp.int32),
            [pltpu.VMEM((CH, D), jnp.float32) for _ in range(NBUF)],
            pltpu.VMEM((_L,), jnp.int32),
            [pltpu.SemaphoreType.DMA for _ in range(NBUF)],
            [pltpu.SemaphoreType.DMA for _ in range(NBUF)],
        ],
    )
    def body(x_hbm, dur_hbm, out_hbm, mel_hbm, dur_v, cnt_v, idx_v, rows,
             mel_v, gsem, wsem):
        wid = lax.axis_index("s") * NC + lax.axis_index("c")
        b = wid // SLOTS
        slot = wid % SLOTS
        f0 = slot * F

        pltpu.sync_copy(dur_hbm.at[b], dur_v)

        zeros_i = jnp.zeros((_L,), jnp.int32)
        for i in range(F // _L):
            cnt_v[pl.ds(i * _L, _L)] = zeros_i

        ones_i = jnp.ones((_L,), jnp.int32)
        carry = jnp.int32(0)
        prefix = jnp.int32(0)
        for i in range(T // _L):
            v = dur_v[pl.ds(i * _L, _L)]
            s = jnp.cumsum(v) + carry
            carry = carry + jnp.sum(v)
            prefix = prefix + jnp.sum((s < f0).astype(jnp.int32))
            in_win = (s >= f0) & (s < f0 + F)
            ci = jnp.clip(s - f0, 0, F - 1)
            plsc.addupdate_scatter(cnt_v, [ci], ones_i, mask=in_win)
        total = carry

        base_row = b * T
        carry2 = prefix
        for i in range(F // _L):
            v = cnt_v[pl.ds(i * _L, _L)]
            j = jnp.cumsum(v) + carry2
            carry2 = carry2 + jnp.sum(v)
            jc = jnp.minimum(j, T - 1)
            idx_v[i // (CH // _L), pl.ds((i % (CH // _L)) * _L, _L)] = (
                jc + base_row)

        @pl.when(slot == 0)
        def _():
            mel_v[...] = jnp.full((_L,), jnp.maximum(total, 1), jnp.int32)
            pltpu.sync_copy(mel_v, mel_hbm.at[b])

        def nv_of(c):
            return jnp.clip(total - (f0 + c * CH), 0, CH)

        def gather(c, p):
            return pltpu.make_async_copy(
                x_hbm.at[idx_v.at[c]], rows[p], gsem[p])

        def writeback(c, p):
            return pltpu.make_async_copy(
                rows[p], out_hbm.at[b, pl.ds(f0 + c * CH, CH)], wsem[p])

        zeros_f = jnp.zeros((_L,), jnp.float32)

        @pl.when(nv_of(0) > 0)
        def _():
            gather(0, 0).start()

        for c in range(NCH):
            p = c % NBUF
            nv = nv_of(c)

            @pl.when(nv > 0)
            def _():
                gather(c, p).wait()

            def zero_row(r, carry_):
                for i in range(D // _L):
                    rows[p][r, pl.ds(i * _L, _L)] = zeros_f
                return carry_

            lax.fori_loop(nv, CH, zero_row, 0)
            writeback(c, p).start()

            if c + 1 < NCH:
                q = (c + 1) % NBUF
                if c + 1 >= NBUF:
                    writeback(c + 1 - NBUF, q).wait()

                @pl.when(nv_of(c + 1) > 0)
                def _():
                    gather(c + 1, q).start()

        for c in range(max(0, NCH - NBUF), NCH):
            writeback(c, c % NBUF).wait()

    return body(x2d, duration)


def kernel(x, duration, max_len):
    B, T, D = x.shape
    out, mel2d = _length_regulate(
        x.reshape(B * T, D), duration.astype(jnp.int32), B, T, D)
    return out, mel2d[:, 0]


# speedup vs baseline: 31.3566x; 1.1093x over previous
import functools

import jax
import jax.numpy as jnp
from jax import lax
from jax.experimental import pallas as pl
from jax.experimental.pallas import tpu as pltpu
from jax.experimental.pallas import tpu_sc as plsc

_L = 16


@functools.partial(jax.jit, static_argnums=(2, 3, 4))
def _length_regulate(x2d, duration, B, T, D):
    ML = 2048
    info = plsc.get_sparse_core_info()
    NC, NS = info.num_cores, info.num_subcores
    NW = NC * NS
    SLOTS = NW // B
    F = ML // SLOTS
    CH = 32
    NCH = F // CH
    NBUF = 4

    mesh = plsc.VectorSubcoreMesh(core_axis_name="c", subcore_axis_name="s")

    @functools.partial(
        pl.kernel,
        mesh=mesh,
        compiler_params=pltpu.CompilerParams(needs_layout_passes=False),
        out_type=[
            jax.ShapeDtypeStruct((B, ML, D), jnp.float32),
            jax.ShapeDtypeStruct((B, _L), jnp.int32),
        ],
        scratch_types=[
            pltpu.VMEM((T,), jnp.int32),
            pltpu.VMEM((F,), jnp.int32),
            pltpu.VMEM((NCH, CH), jnp.int32),
            [pltpu.VMEM((CH, D), jnp.float32) for _ in range(NBUF)],
            pltpu.VMEM((_L,), jnp.int32),
            [pltpu.SemaphoreType.DMA for _ in range(NBUF)],
            [pltpu.SemaphoreType.DMA for _ in range(NBUF)],
        ],
    )
    def body(x_hbm, dur_hbm, out_hbm, mel_hbm, dur_v, cnt_v, idx_v, rows,
             mel_v, gsem, wsem):
        wid = lax.axis_index("s") * NC + lax.axis_index("c")
        b = wid // SLOTS
        slot = wid % SLOTS
        f0 = slot * F

        pltpu.sync_copy(dur_hbm.at[b], dur_v)

        zeros_i = jnp.zeros((_L,), jnp.int32)
        for i in range(F // _L):
            cnt_v[pl.ds(i * _L, _L)] = zeros_i

        ones_i = jnp.ones((_L,), jnp.int32)
        carry = jnp.int32(0)
        prefix = jnp.int32(0)
        for i in range(T // _L):
            v = dur_v[pl.ds(i * _L, _L)]
            s = jnp.cumsum(v) + carry
            carry = carry + jnp.sum(v)
            prefix = prefix + jnp.sum((s < f0).astype(jnp.int32))
            in_win = (s >= f0) & (s < f0 + F)
            ci = jnp.clip(s - f0, 0, F - 1)
            plsc.addupdate_scatter(cnt_v, [ci], ones_i, mask=in_win)
        total = carry

        base_row = b * T
        carry2 = prefix
        for i in range(F // _L):
            v = cnt_v[pl.ds(i * _L, _L)]
            j = jnp.cumsum(v) + carry2
            carry2 = carry2 + jnp.sum(v)
            jc = jnp.minimum(j, T - 1)
            idx_v[i // (CH // _L), pl.ds((i % (CH // _L)) * _L, _L)] = (
                jc + base_row)

        @pl.when(slot == 0)
        def _():
            mel_v[...] = jnp.full((_L,), jnp.maximum(total, 1), jnp.int32)
            pltpu.sync_copy(mel_v, mel_hbm.at[b])

        def nv_of(c):
            return jnp.clip(total - (f0 + c * CH), 0, CH)

        def gather(c, p):
            return pltpu.make_async_copy(
                x_hbm.at[idx_v.at[c]], rows[p], gsem[p])

        def writeback(c, p):
            return pltpu.make_async_copy(
                rows[p], out_hbm.at[b, pl.ds(f0 + c * CH, CH)], wsem[p])

        zeros_f = jnp.zeros((_L,), jnp.float32)

        @pl.when(nv_of(0) > 0)
        def _():
            gather(0, 0).start()

        def super_step(k, carry_):
            for p in range(NBUF):
                c = k * NBUF + p
                pn = (p + 1) % NBUF
                cp = c + 1 - NBUF

                @pl.when(cp >= 0)
                def _():
                    writeback(jnp.maximum(cp, 0), pn).wait()

                @pl.when((c + 1 < NCH) & (nv_of(c + 1) > 0))
                def _():
                    gather(jnp.minimum(c + 1, NCH - 1), pn).start()

                nv = nv_of(c)

                @pl.when(nv > 0)
                def _():
                    gather(c, p).wait()

                def zero_row(r, cy):
                    for i in range(D // _L):
                        rows[p][r, pl.ds(i * _L, _L)] = zeros_f
                    return cy

                lax.fori_loop(nv, CH, zero_row, 0)
                writeback(c, p).start()
            return carry_

        lax.fori_loop(0, NCH // NBUF, super_step, 0)

        for c in range(NCH - NBUF + 1, NCH):
            writeback(c, c % NBUF).wait()

    return body(x2d, duration)


def kernel(x, duration, max_len):
    B, T, D = x.shape
    out, mel2d = _length_regulate(
        x.reshape(B * T, D), duration.astype(jnp.int32), B, T, D)
    return out, mel2d[:, 0]


# speedup vs baseline: 31.3631x; 1.0002x over previous
import functools

import jax
import jax.numpy as jnp
from jax import lax
from jax.experimental import pallas as pl
from jax.experimental.pallas import tpu as pltpu
from jax.experimental.pallas import tpu_sc as plsc

_L = 16


@functools.partial(jax.jit, static_argnums=(2, 3, 4))
def _length_regulate(x2d, duration, B, T, D):
    ML = 2048
    info = plsc.get_sparse_core_info()
    NC, NS = info.num_cores, info.num_subcores
    NW = NC * NS
    SLOTS = NW // B
    F = ML // SLOTS
    CH = 32
    NCH = F // CH
    NBUF = 4

    mesh = plsc.VectorSubcoreMesh(core_axis_name="c", subcore_axis_name="s")

    @functools.partial(
        pl.kernel,
        mesh=mesh,
        compiler_params=pltpu.CompilerParams(needs_layout_passes=False),
        out_type=[
            jax.ShapeDtypeStruct((B, ML, D), jnp.float32),
            jax.ShapeDtypeStruct((B, _L), jnp.int32),
        ],
        scratch_types=[
            pltpu.VMEM((T,), jnp.int32),
            pltpu.VMEM((F,), jnp.int32),
            pltpu.VMEM((NCH, CH), jnp.int32),
            [pltpu.VMEM((CH, D), jnp.float32) for _ in range(NBUF)],
            pltpu.VMEM((_L,), jnp.int32),
            [pltpu.SemaphoreType.DMA for _ in range(NBUF)],
            [pltpu.SemaphoreType.DMA for _ in range(NBUF)],
        ],
    )
    def body(x_hbm, dur_hbm, out_hbm, mel_hbm, dur_v, cnt_v, idx_v, rows,
             mel_v, gsem, wsem):
        wid = lax.axis_index("s") * NC + lax.axis_index("c")
        b = wid // SLOTS
        slot = wid % SLOTS
        f0 = slot * F

        pltpu.sync_copy(dur_hbm.at[b], dur_v)

        zeros_i = jnp.zeros((_L,), jnp.int32)

        def zero_hist(i, cy):
            cnt_v[pl.ds(i * _L, _L)] = zeros_i
            return cy

        lax.fori_loop(0, F // _L, zero_hist, 0)

        ones_i = jnp.ones((_L,), jnp.int32)
        carry = jnp.int32(0)
        prefix = jnp.int32(0)
        for i in range(T // _L):
            v = dur_v[pl.ds(i * _L, _L)]
            s = jnp.cumsum(v) + carry
            carry = carry + jnp.sum(v)
            prefix = prefix + jnp.sum((s < f0).astype(jnp.int32))
            in_win = (s >= f0) & (s < f0 + F)
            ci = jnp.clip(s - f0, 0, F - 1)
            plsc.addupdate_scatter(cnt_v, [ci], ones_i, mask=in_win)
        total = carry
        prefix = prefix

        base_row = b * T

        def scan_hist(i, carry2):
            v = cnt_v[pl.ds(i * _L, _L)]
            j = jnp.cumsum(v) + carry2
            jc = jnp.minimum(j, T - 1)
            idx_v[i // (CH // _L), pl.ds((i % (CH // _L)) * _L, _L)] = (
                jc + base_row)
            return carry2 + jnp.sum(v)

        lax.fori_loop(0, F // _L, scan_hist, prefix)

        @pl.when(slot == 0)
        def _():
            mel_v[...] = jnp.full((_L,), jnp.maximum(total, 1), jnp.int32)
            pltpu.sync_copy(mel_v, mel_hbm.at[b])

        def nv_of(c):
            return jnp.clip(total - (f0 + c * CH), 0, CH)

        def gather(c, p):
            return pltpu.make_async_copy(
                x_hbm.at[idx_v.at[c]], rows[p], gsem[p])

        def writeback(c, p):
            return pltpu.make_async_copy(
                rows[p], out_hbm.at[b, pl.ds(f0 + c * CH, CH)], wsem[p])

        zeros_f = jnp.zeros((_L,), jnp.float32)

        @pl.when(nv_of(0) > 0)
        def _():
            gather(0, 0).start()

        def super_step(k, carry_):
            for p in range(NBUF):
                c = k * NBUF + p
                pn = (p + 1) % NBUF
                cp = c + 1 - NBUF

                @pl.when(cp >= 0)
                def _():
                    writeback(jnp.maximum(cp, 0), pn).wait()

                @pl.when((c + 1 < NCH) & (nv_of(c + 1) > 0))
                def _():
                    gather(jnp.minimum(c + 1, NCH - 1), pn).start()

                nv = nv_of(c)

                @pl.when(nv > 0)
                def _():
                    gather(c, p).wait()

                def zero_row(r, cy):
                    for i in range(D // _L):
                        rows[p][r, pl.ds(i * _L, _L)] = zeros_f
                    return cy

                lax.fori_loop(nv, CH, zero_row, 0)
                writeback(c, p).start()
            return carry_

        lax.fori_loop(0, NCH // NBUF, super_step, 0)

        for c in range(NCH - NBUF + 1, NCH):
            writeback(c, c % NBUF).wait()

    return body(x2d, duration)


def kernel(x, duration, max_len):
    B, T, D = x.shape
    out, mel2d = _length_regulate(
        x.reshape(B * T, D), duration.astype(jnp.int32), B, T, D)
    return out, mel2d[:, 0]


# speedup vs baseline: 32.7027x; 1.0427x over previous
import functools

import jax
import jax.numpy as jnp
from jax import lax
from jax.experimental import pallas as pl
from jax.experimental.pallas import tpu as pltpu
from jax.experimental.pallas import tpu_sc as plsc

_L = 16


@functools.partial(jax.jit, static_argnums=(2, 3, 4))
def _length_regulate(x2d, duration, B, T, D):
    ML = 2048
    info = plsc.get_sparse_core_info()
    NC, NS = info.num_cores, info.num_subcores
    NW = NC * NS
    SLOTS = NW // B
    F = ML // SLOTS
    CH = 32
    NCH = F // CH
    NBUF = 4

    mesh = plsc.VectorSubcoreMesh(core_axis_name="c", subcore_axis_name="s")

    @functools.partial(
        pl.kernel,
        mesh=mesh,
        compiler_params=pltpu.CompilerParams(needs_layout_passes=False),
        out_type=[
            jax.ShapeDtypeStruct((B, ML, D), jnp.float32),
            jax.ShapeDtypeStruct((B, _L), jnp.int32),
        ],
        scratch_types=[
            pltpu.VMEM((T,), jnp.int32),
            pltpu.VMEM((F,), jnp.int32),
            pltpu.VMEM((NCH, CH), jnp.int32),
            [pltpu.VMEM((CH, D), jnp.float32) for _ in range(NBUF)],
            pltpu.VMEM((_L,), jnp.int32),
            [pltpu.SemaphoreType.DMA for _ in range(NBUF)],
            [pltpu.SemaphoreType.DMA for _ in range(NBUF)],
        ],
    )
    def body(x_hbm, dur_hbm, out_hbm, mel_hbm, dur_v, cnt_v, idx_v, rows,
             mel_v, gsem, wsem):
        wid = lax.axis_index("s") * NC + lax.axis_index("c")
        b = wid // SLOTS
        slot = wid % SLOTS
        f0 = slot * F

        pltpu.sync_copy(dur_hbm.at[b], dur_v)

        zeros_i = jnp.zeros((_L,), jnp.int32)

        def zero_hist(i, cy):
            cnt_v[pl.ds(i * _L, _L)] = zeros_i
            return cy

        lax.fori_loop(0, F // _L, zero_hist, 0)

        ones_i = jnp.ones((_L,), jnp.int32)
        carry = jnp.int32(0)
        prefix = jnp.int32(0)
        for i in range(T // _L):
            v = dur_v[pl.ds(i * _L, _L)]
            s = jnp.cumsum(v) + carry
            carry = carry + jnp.sum(v)
            prefix = prefix + jnp.sum((s < f0).astype(jnp.int32))
            in_win = (s >= f0) & (s < f0 + F)
            ci = jnp.clip(s - f0, 0, F - 1)
            plsc.addupdate_scatter(cnt_v, [ci], ones_i, mask=in_win)
        total = carry
        prefix = prefix

        base_row = b * T

        def scan_hist(i, carry2):
            v = cnt_v[pl.ds(i * _L, _L)]
            j = jnp.cumsum(v) + carry2
            jc = jnp.minimum(j, T - 1)
            idx_v[i // (CH // _L), pl.ds((i % (CH // _L)) * _L, _L)] = (
                jc + base_row)
            return carry2 + jnp.sum(v)

        lax.fori_loop(0, F // _L, scan_hist, prefix)

        @pl.when(slot == 0)
        def _():
            mel_v[...] = jnp.full((_L,), jnp.maximum(total, 1), jnp.int32)
            pltpu.sync_copy(mel_v, mel_hbm.at[b])

        def nv_of(c):
            return jnp.clip(total - (f0 + c * CH), 0, CH)

        def gather(c, p):
            return pltpu.make_async_copy(
                x_hbm.at[idx_v.at[c]], rows[p], gsem[p])

        def writeback(c, p):
            return pltpu.make_async_copy(
                rows[p], out_hbm.at[b, pl.ds(f0 + c * CH, CH)], wsem[p])

        zeros_f = jnp.zeros((_L,), jnp.float32)
        LOOK = 2

        for c in range(LOOK):
            @pl.when(nv_of(c) > 0)
            def _(c=c):
                gather(c, c % NBUF).start()

        def super_step(k, carry_):
            for p in range(NBUF):
                c = k * NBUF + p
                pn = (p + LOOK) % NBUF
                cp = c + LOOK - NBUF

                @pl.when(cp >= 0)
                def _():
                    writeback(jnp.maximum(cp, 0), pn).wait()

                @pl.when((c + LOOK < NCH) & (nv_of(c + LOOK) > 0))
                def _():
                    gather(jnp.minimum(c + LOOK, NCH - 1), pn).start()

                nv = nv_of(c)

                @pl.when(nv > 0)
                def _():
                    gather(c, p).wait()

                def zero_row(r, cy):
                    for i in range(D // _L):
                        rows[p][r, pl.ds(i * _L, _L)] = zeros_f
                    return cy

                lax.fori_loop(nv, CH, zero_row, 0)
                writeback(c, p).start()
            return carry_

        lax.fori_loop(0, NCH // NBUF, super_step, 0)

        for c in range(NCH - NBUF + LOOK, NCH):
            writeback(c, c % NBUF).wait()

    return body(x2d, duration)


def kernel(x, duration, max_len):
    B, T, D = x.shape
    out, mel2d = _length_regulate(
        x.reshape(B * T, D), duration.astype(jnp.int32), B, T, D)
    return out, mel2d[:, 0]


# speedup vs baseline: 32.7172x; 1.0004x over previous
import functools

import jax
import jax.numpy as jnp
from jax import lax
from jax.experimental import pallas as pl
from jax.experimental.pallas import tpu as pltpu
from jax.experimental.pallas import tpu_sc as plsc

_L = 16


@functools.partial(jax.jit, static_argnums=(2, 3, 4))
def _length_regulate(x2d, duration, B, T, D):
    ML = 2048
    info = plsc.get_sparse_core_info()
    NC, NS = info.num_cores, info.num_subcores
    NW = NC * NS
    SLOTS = NW // B
    F = ML // SLOTS
    CH = 32
    NCH = F // CH
    NBUF = 4
    assert NW % B == 0 and ML % SLOTS == 0 and F % CH == 0
    assert NCH % NBUF == 0 and T % _L == 0 and F % _L == 0 and D % _L == 0

    mesh = plsc.VectorSubcoreMesh(core_axis_name="c", subcore_axis_name="s")

    @functools.partial(
        pl.kernel,
        mesh=mesh,
        compiler_params=pltpu.CompilerParams(needs_layout_passes=False),
        out_type=[
            jax.ShapeDtypeStruct((B, ML, D), jnp.float32),
            jax.ShapeDtypeStruct((B, _L), jnp.int32),
        ],
        scratch_types=[
            pltpu.VMEM((T,), jnp.int32),
            pltpu.VMEM((F,), jnp.int32),
            pltpu.VMEM((NCH, CH), jnp.int32),
            [pltpu.VMEM((CH, D), jnp.float32) for _ in range(NBUF)],
            pltpu.VMEM((CH, D), jnp.float32),
            pltpu.VMEM((_L,), jnp.int32),
            [pltpu.SemaphoreType.DMA for _ in range(NBUF)],
            [pltpu.SemaphoreType.DMA for _ in range(NBUF)],
            pltpu.SemaphoreType.DMA,
        ],
    )
    def body(x_hbm, dur_hbm, out_hbm, mel_hbm, dur_v, cnt_v, idx_v, rows,
             zrow_v, mel_v, gsem, wsem, dsem):
        wid = lax.axis_index("s") * NC + lax.axis_index("c")
        b = wid // SLOTS
        slot = wid % SLOTS
        f0 = slot * F

        dur_copy = pltpu.make_async_copy(dur_hbm.at[b], dur_v, dsem)
        dur_copy.start()

        zeros_i = jnp.zeros((_L,), jnp.int32)
        zeros_f = jnp.zeros((_L,), jnp.float32)

        def zero_hist(i, cy):
            cnt_v[pl.ds(i * _L, _L)] = zeros_i
            return cy

        lax.fori_loop(0, F // _L, zero_hist, 0)

        def zero_chunk(r, cy):
            for i in range(D // _L):
                zrow_v[r, pl.ds(i * _L, _L)] = zeros_f
            return cy

        lax.fori_loop(0, CH, zero_chunk, 0)
        dur_copy.wait()

        ones_i = jnp.ones((_L,), jnp.int32)
        carry = jnp.int32(0)
        prefix = jnp.int32(0)
        for i in range(T // _L):
            v = dur_v[pl.ds(i * _L, _L)]
            s = jnp.cumsum(v) + carry
            carry = carry + jnp.sum(v)
            prefix = prefix + jnp.sum((s < f0).astype(jnp.int32))
            in_win = (s >= f0) & (s < f0 + F)
            ci = jnp.clip(s - f0, 0, F - 1)
            plsc.addupdate_scatter(cnt_v, [ci], ones_i, mask=in_win)
        total = carry

        base_row = b * T

        def scan_hist(i, carry2):
            v = cnt_v[pl.ds(i * _L, _L)]
            j = jnp.cumsum(v) + carry2
            jc = jnp.minimum(j, T - 1)
            idx_v[i // (CH // _L), pl.ds((i % (CH // _L)) * _L, _L)] = (
                jc + base_row)
            return carry2 + jnp.sum(v)

        lax.fori_loop(0, F // _L, scan_hist, prefix)

        @pl.when(slot == 0)
        def _():
            mel_v[...] = jnp.full((_L,), jnp.maximum(total, 1), jnp.int32)
            pltpu.sync_copy(mel_v, mel_hbm.at[b])

        def nv_of(c):
            return jnp.clip(total - (f0 + c * CH), 0, CH)

        def gather(c, p):
            return pltpu.make_async_copy(
                x_hbm.at[idx_v.at[c]], rows[p], gsem[p])

        def writeback(c, p):
            return pltpu.make_async_copy(
                rows[p], out_hbm.at[b, pl.ds(f0 + c * CH, CH)], wsem[p])

        LOOK = 2

        for c in range(LOOK):
            @pl.when(nv_of(c) > 0)
            def _(c=c):
                gather(c, c % NBUF).start()

        def super_step(k, carry_):
            for p in range(NBUF):
                c = k * NBUF + p
                pn = (p + LOOK) % NBUF
                cp = c + LOOK - NBUF

                @pl.when(cp >= 0)
                def _():
                    writeback(jnp.maximum(cp, 0), pn).wait()

                @pl.when((c + LOOK < NCH) & (nv_of(c + LOOK) > 0))
                def _():
                    gather(jnp.minimum(c + LOOK, NCH - 1), pn).start()

                nv = nv_of(c)

                @pl.when(nv > 0)
                def _():
                    gather(c, p).wait()

                    @pl.when(nv < CH)
                    def _():
                        def zero_row(r, cy):
                            for i in range(D // _L):
                                rows[p][r, pl.ds(i * _L, _L)] = zeros_f
                            return cy

                        lax.fori_loop(nv, CH, zero_row, 0)

                    writeback(c, p).start()

                @pl.when(nv == 0)
                def _():
                    pltpu.make_async_copy(
                        zrow_v, out_hbm.at[b, pl.ds(f0 + c * CH, CH)],
                        wsem[p]).start()
            return carry_

        lax.fori_loop(0, NCH // NBUF, super_step, 0)

        for c in range(NCH - NBUF + LOOK, NCH):
            writeback(c, c % NBUF).wait()

    return body(x2d, duration)


def kernel(x, duration, max_len):
    B, T, D = x.shape
    out, mel2d = _length_regulate(
        x.reshape(B * T, D), duration.astype(jnp.int32), B, T, D)
    return out, mel2d[:, 0]


# speedup vs baseline: 32.9414x; 1.0069x over previous
import functools

import jax
import jax.numpy as jnp
from jax import lax
from jax.experimental import pallas as pl
from jax.experimental.pallas import tpu as pltpu
from jax.experimental.pallas import tpu_sc as plsc

_L = 16


@functools.partial(jax.jit, static_argnums=(2, 3, 4))
def _length_regulate(x2d, duration, B, T, D):
    ML = 2048
    info = plsc.get_sparse_core_info()
    NC, NS = info.num_cores, info.num_subcores
    NW = NC * NS
    SLOTS = NW // B
    F = ML // SLOTS
    CH = 32
    NCH = F // CH
    NBUF = 4
    assert NW % B == 0 and ML % SLOTS == 0 and F % CH == 0
    assert NCH % NBUF == 0 and T % _L == 0 and F % _L == 0 and D % _L == 0

    mesh = plsc.VectorSubcoreMesh(core_axis_name="c", subcore_axis_name="s")

    @functools.partial(
        pl.kernel,
        mesh=mesh,
        compiler_params=pltpu.CompilerParams(needs_layout_passes=False),
        out_type=[
            jax.ShapeDtypeStruct((B, ML, D), jnp.float32),
            jax.ShapeDtypeStruct((B, _L), jnp.int32),
        ],
        scratch_types=[
            pltpu.VMEM((T,), jnp.int32),
            pltpu.VMEM((F,), jnp.int32),
            pltpu.VMEM((NCH, CH), jnp.int32),
            [pltpu.VMEM((CH, D), jnp.float32) for _ in range(NBUF)],
            pltpu.VMEM((CH, D), jnp.float32),
            pltpu.VMEM((_L,), jnp.int32),
            [pltpu.SemaphoreType.DMA for _ in range(NBUF)],
            [pltpu.SemaphoreType.DMA for _ in range(NBUF)],
            pltpu.SemaphoreType.DMA,
        ],
    )
    def body(x_hbm, dur_hbm, out_hbm, mel_hbm, dur_v, cnt_v, idx_v, rows,
             zrow_v, mel_v, gsem, wsem, dsem):
        wid = lax.axis_index("s") * NC + lax.axis_index("c")
        b = wid // SLOTS
        slot = wid % SLOTS
        f0 = slot * F

        dur_copy = pltpu.make_async_copy(dur_hbm.at[b], dur_v, dsem)
        dur_copy.start()

        zeros_i = jnp.zeros((_L,), jnp.int32)
        zeros_f = jnp.zeros((_L,), jnp.float32)

        def zero_hist(i, cy):
            cnt_v[pl.ds(i * _L, _L)] = zeros_i
            return cy

        lax.fori_loop(0, F // _L, zero_hist, 0)

        def zero_chunk(r, cy):
            for i in range(D // _L):
                zrow_v[r, pl.ds(i * _L, _L)] = zeros_f
            return cy

        lax.fori_loop(0, CH, zero_chunk, 0)
        dur_copy.wait()

        ones_i = jnp.ones((_L,), jnp.int32)
        carry = jnp.int32(0)
        prefix = jnp.int32(0)
        for i in range(T // _L):
            v = dur_v[pl.ds(i * _L, _L)]
            s = jnp.cumsum(v) + carry
            carry = carry + jnp.sum(v)
            prefix = prefix + jnp.sum((s < f0).astype(jnp.int32))
            in_win = (s >= f0) & (s < f0 + F)
            ci = jnp.clip(s - f0, 0, F - 1)
            plsc.addupdate_scatter(cnt_v, [ci], ones_i, mask=in_win)
        total = carry

        base_row = b * T

        def scan_hist(i, carry2):
            v = cnt_v[pl.ds(i * _L, _L)]
            j = jnp.cumsum(v) + carry2
            jc = jnp.minimum(j, T - 1)
            idx_v[i // (CH // _L), pl.ds((i % (CH // _L)) * _L, _L)] = (
                jc + base_row)
            return carry2 + jnp.sum(v)

        def nv_of(c):
            return jnp.clip(total - (f0 + c * CH), 0, CH)

        def gather(c, p):
            return pltpu.make_async_copy(
                x_hbm.at[idx_v.at[c]], rows[p], gsem[p])

        def writeback(c, p):
            return pltpu.make_async_copy(
                rows[p], out_hbm.at[b, pl.ds(f0 + c * CH, CH)], wsem[p])

        LOOK = 2
        PEEL = LOOK * (CH // _L)

        carry2 = lax.fori_loop(0, PEEL, scan_hist, prefix)

        for c in range(LOOK):
            @pl.when(nv_of(c) > 0)
            def _(c=c):
                gather(c, c % NBUF).start()

        lax.fori_loop(PEEL, F // _L, scan_hist, carry2)

        @pl.when(slot == 0)
        def _():
            mel_v[...] = jnp.full((_L,), jnp.maximum(total, 1), jnp.int32)
            pltpu.sync_copy(mel_v, mel_hbm.at[b])

        def super_step(k, carry_):
            for p in range(NBUF):
                c = k * NBUF + p
                pn = (p + LOOK) % NBUF
                cp = c + LOOK - NBUF

                @pl.when(cp >= 0)
                def _():
                    writeback(jnp.maximum(cp, 0), pn).wait()

                @pl.when((c + LOOK < NCH) & (nv_of(c + LOOK) > 0))
                def _():
                    gather(jnp.minimum(c + LOOK, NCH - 1), pn).start()

                nv = nv_of(c)

                @pl.when(nv > 0)
                def _():
                    gather(c, p).wait()

                    @pl.when(nv < CH)
                    def _():
                        def zero_row(r, cy):
                            for i in range(D // _L):
                                rows[p][r, pl.ds(i * _L, _L)] = zeros_f
                            return cy

                        lax.fori_loop(nv, CH, zero_row, 0)

                    writeback(c, p).start()

                @pl.when(nv == 0)
                def _():
                    pltpu.make_async_copy(
                        zrow_v, out_hbm.at[b, pl.ds(f0 + c * CH, CH)],
                        wsem[p]).start()
            return carry_

        lax.fori_loop(0, NCH // NBUF, super_step, 0)

        for c in range(NCH - NBUF + LOOK, NCH):
            writeback(c, c % NBUF).wait()

    return body(x2d, duration)


def kernel(x, duration, max_len):
    B, T, D = x.shape
    out, mel2d = _length_regulate(
        x.reshape(B * T, D), duration.astype(jnp.int32), B, T, D)
    return out, mel2d[:, 0]
